# baseline (speedup 1.0000x reference)
import functools

import jax
import jax.numpy as jnp
from jax import lax
from jax.experimental import pallas as pl
from jax.experimental.pallas import tpu as pltpu
from jax.experimental.pallas import tpu_sc as plsc

V = 100000
D = 64
K = 10
CTX = 10
B = 16384
LP = 128
NC = 2
NS = 16
NW = NC * NS
BPW = B // NW
C = 32
N_CHUNKS = BPW // C
S = K + 1


def _tree_sum(vals):
    vals = list(vals)
    while len(vals) > 1:
        nxt = [a + b for a, b in zip(vals[0::2], vals[1::2])]
        if len(vals) % 2:
            nxt.append(vals[-1])
        vals = nxt
    return vals[0]


def _sc_loss_kernel(i_emb, o_emb, tgt_hbm, ctx_hbm, neg_hbm, out_hbm,
                    tgt_idx, ctx_stage, neg_stage, ctx_idx, neg_idx,
                    tgt_rows, ctx_rows, neg_rows, pbuf, acc_buf,
                    sem_i, sem_t, sem_c, sem_n):
    wid = lax.axis_index("s") * NC + lax.axis_index("c")
    base = wid * BPW

    lane = lax.iota(jnp.int32, 16)
    NG = C * CTX // 16
    pos_r, pos_c = [], []
    for g in range(NG):
        p = lane + g * 16
        c = p // CTX
        pos_r.append(c)
        pos_c.append(p - c * CTX)

    def fire(t, b):
        row0 = base + t * C
        sl = pl.ds(row0, C)
        pltpu.async_copy(tgt_hbm.at[sl], tgt_idx.at[b], sem_i.at[b])
        pltpu.async_copy(ctx_hbm.at[sl, :], ctx_stage.at[b], sem_i.at[b])
        pltpu.async_copy(neg_hbm.at[sl, :], neg_stage.at[b], sem_i.at[b])
        pltpu.make_async_copy(tgt_hbm.at[sl], tgt_idx.at[b], sem_i.at[b]).wait()
        pltpu.make_async_copy(
            ctx_hbm.at[sl, :], ctx_stage.at[b], sem_i.at[b]).wait()
        pltpu.make_async_copy(
            neg_hbm.at[sl, :], neg_stage.at[b], sem_i.at[b]).wait()
        for g in range(NG):
            ctx_idx[b, pl.ds(g * 16, 16)] = plsc.load_gather(
                ctx_stage.at[b], [pos_r[g], pos_c[g]])
            neg_idx[b, pl.ds(g * 16, 16)] = plsc.load_gather(
                neg_stage.at[b], [pos_r[g], pos_c[g]])
        pltpu.async_copy(o_emb.at[tgt_idx.at[b]], tgt_rows.at[b], sem_t.at[b])
        pltpu.async_copy(i_emb.at[ctx_idx.at[b]], ctx_rows.at[b], sem_c.at[b])
        pltpu.async_copy(o_emb.at[neg_idx.at[b]], neg_rows.at[b], sem_n.at[b])

    def drain(b):
        pltpu.make_async_copy(
            o_emb.at[tgt_idx.at[b]], tgt_rows.at[b], sem_t.at[b]).wait()
        pltpu.make_async_copy(
            i_emb.at[ctx_idx.at[b]], ctx_rows.at[b], sem_c.at[b]).wait()
        pltpu.make_async_copy(
            o_emb.at[neg_idx.at[b]], neg_rows.at[b], sem_n.at[b]).wait()

    lane_sel = jnp.where(lane < S, lane, 0)
    sgn = jnp.where(lane == 0, 1.0, -1.0)
    valid = lane < S
    col_idx = [jnp.full((16,), j, jnp.int32) for j in range(16)]

    def compute(b, acc0):
        def dotf32(row_ref, r, avg):
            p = [row_ref[b, r, pl.ds(q * 32, 32)] * avg[q]
                 for q in range(D // 32)]
            p = _tree_sum(p)
            lo, hi = plsc.unpack(p, format=plsc.PackFormat.INTERLEAVED)
            return lo + hi

        def row_body(c, acc):
            rc = c * CTX
            avg = []
            for q in range(D // 32):
                a = _tree_sum(
                    [ctx_rows[b, rc + j, pl.ds(q * 32, 32)]
                     for j in range(CTX)])
                avg.append(a * jnp.bfloat16(1.0 / CTX))
            pbuf[0, :] = dotf32(tgt_rows, c, avg)
            rn = c * K
            for k in range(K):
                pbuf[k + 1, :] = dotf32(neg_rows, rn + k, avg)
            s = _tree_sum(
                [plsc.load_gather(pbuf, [lane_sel, col_idx[j]])
                 for j in range(16)])
            x = sgn * s
            y = jnp.exp(-jnp.abs(x))
            t = y / (y + 2.0)
            u = t * t
            poly = 1.0 + u * (1.0 / 3.0 + u * (1.0 / 5.0 + u * (1.0 / 7.0)))
            sp = jnp.maximum(-x, 0.0) + 2.0 * t * poly
            return acc + jnp.where(valid, sp, 0.0)

        return lax.fori_loop(0, C, row_body, acc0)

    fire(0, 0)
    acc = jnp.zeros((16,), jnp.float32)

    def body(i, acc):
        fire(2 * i + 1, 1)
        drain(0)
        acc = compute(0, acc)

        @pl.when(i < N_CHUNKS // 2 - 1)
        def _():
            fire(2 * i + 2, 0)

        drain(1)
        return compute(1, acc)

    acc = lax.fori_loop(0, N_CHUNKS // 2, body, acc)
    acc_buf[...] = acc
    pltpu.sync_copy(acc_buf, out_hbm.at[wid, :])


_sc_loss = functools.partial(
    pl.kernel,
    mesh=plsc.VectorSubcoreMesh(core_axis_name="c", subcore_axis_name="s"),
    compiler_params=pltpu.CompilerParams(
        needs_layout_passes=False, use_tc_tiling_on_sc=False
    ),
    out_type=jax.ShapeDtypeStruct((NW, 16), jnp.float32),
    scratch_types=[
        pltpu.VMEM((2, C), jnp.int32),
        pltpu.VMEM((2, C, LP), jnp.int32),
        pltpu.VMEM((2, C, LP), jnp.int32),
        pltpu.VMEM((2, C * CTX), jnp.int32),
        pltpu.VMEM((2, C * K), jnp.int32),
        pltpu.VMEM((2, C, D), jnp.bfloat16),
        pltpu.VMEM((2, C * CTX, D), jnp.bfloat16),
        pltpu.VMEM((2, C * K, D), jnp.bfloat16),
        pltpu.VMEM((16, 16), jnp.float32),
        pltpu.VMEM((16,), jnp.float32),
        pltpu.SemaphoreType.DMA((2,)),
        pltpu.SemaphoreType.DMA((2,)),
        pltpu.SemaphoreType.DMA((2,)),
        pltpu.SemaphoreType.DMA((2,)),
    ],
)(_sc_loss_kernel)


def _tc_loss_kernel(x_ref, o_ref):
    o_ref[0, 0] = jnp.sum(x_ref[...])


_tc_loss = pl.pallas_call(
    _tc_loss_kernel,
    out_shape=jax.ShapeDtypeStruct((1, 1), jnp.float32),
    out_specs=pl.BlockSpec(memory_space=pltpu.SMEM),
)


def kernel(i_emb, o_emb, target_wids, context_wids, neg_wids):
    i_emb = i_emb.astype(jnp.bfloat16)
    o_emb = o_emb.astype(jnp.bfloat16)
    tgt = target_wids.astype(jnp.int32)
    ctxp = jnp.pad(context_wids.astype(jnp.int32), ((0, 0), (0, LP - CTX)))
    negp = jnp.pad(neg_wids.astype(jnp.int32), ((0, 0), (0, LP - K)))
    partials = _sc_loss(i_emb, o_emb, tgt, ctxp, negp)
    loss = _tc_loss(partials)
    return loss[0, 0]


# speedup vs baseline: 1.3681x; 1.3681x over previous
import functools

import jax
import jax.numpy as jnp
from jax import lax
from jax.experimental import pallas as pl
from jax.experimental.pallas import tpu as pltpu
from jax.experimental.pallas import tpu_sc as plsc

V = 100000
D = 64
K = 10
CTX = 10
B = 16384
LP = 128
NC = 2
NS = 16
NW = NC * NS
BPW = B // NW
C = 32
N_CHUNKS = BPW // C
S = K + 1


def _tree_sum(vals):
    vals = list(vals)
    while len(vals) > 1:
        nxt = [a + b for a, b in zip(vals[0::2], vals[1::2])]
        if len(vals) % 2:
            nxt.append(vals[-1])
        vals = nxt
    return vals[0]


def _sc_loss_kernel(i_emb, o_emb, tgt_hbm, ctx_hbm, neg_hbm, out_hbm,
                    tgt_idx, ctx_stage, neg_stage, ctx_idx, neg_idx,
                    tgt_rows, ctx_rows, neg_rows, pbuf, acc_buf,
                    sem_i, sem_t, sem_c, sem_n):
    wid = lax.axis_index("s") * NC + lax.axis_index("c")
    base = wid * BPW

    lane = lax.iota(jnp.int32, 16)
    NG = C * CTX // 16
    pos_r, pos_c = [], []
    for g in range(NG):
        p = lane + g * 16
        c = p // CTX
        pos_r.append(c)
        pos_c.append(p - c * CTX)

    def fire(t, b):
        row0 = base + t * C
        sl = pl.ds(row0, C)
        pltpu.async_copy(tgt_hbm.at[sl], tgt_idx.at[b], sem_i.at[b])
        pltpu.async_copy(ctx_hbm.at[sl, :], ctx_stage.at[b], sem_i.at[b])
        pltpu.async_copy(neg_hbm.at[sl, :], neg_stage.at[b], sem_i.at[b])
        pltpu.make_async_copy(tgt_hbm.at[sl], tgt_idx.at[b], sem_i.at[b]).wait()
        pltpu.make_async_copy(
            ctx_hbm.at[sl, :], ctx_stage.at[b], sem_i.at[b]).wait()
        pltpu.make_async_copy(
            neg_hbm.at[sl, :], neg_stage.at[b], sem_i.at[b]).wait()
        for g in range(NG):
            ctx_idx[b, pl.ds(g * 16, 16)] = plsc.load_gather(
                ctx_stage.at[b], [pos_r[g], pos_c[g]])
            neg_idx[b, pl.ds(g * 16, 16)] = plsc.load_gather(
                neg_stage.at[b], [pos_r[g], pos_c[g]])
        pltpu.async_copy(o_emb.at[tgt_idx.at[b]], tgt_rows.at[b], sem_t.at[b])
        pltpu.async_copy(i_emb.at[ctx_idx.at[b]], ctx_rows.at[b], sem_c.at[b])
        pltpu.async_copy(o_emb.at[neg_idx.at[b]], neg_rows.at[b], sem_n.at[b])

    def drain(b):
        pltpu.make_async_copy(
            o_emb.at[tgt_idx.at[b]], tgt_rows.at[b], sem_t.at[b]).wait()
        pltpu.make_async_copy(
            i_emb.at[ctx_idx.at[b]], ctx_rows.at[b], sem_c.at[b]).wait()
        pltpu.make_async_copy(
            o_emb.at[neg_idx.at[b]], neg_rows.at[b], sem_n.at[b]).wait()

    lane_sel = jnp.where(lane < S, lane, 0)
    sgn = jnp.where(lane == 0, 1.0, -1.0)
    valid = lane < S
    col_idx = [jnp.full((16,), j, jnp.int32) for j in range(16)]

    def compute(b, acc0):
        def row_body(c, acc):
            rc = c * CTX
            avg = []
            for q in range(D // 16):
                a = _tree_sum(
                    [ctx_rows[b, rc + j, pl.ds(q * 16, 16)]
                     for j in range(CTX)])
                avg.append(a * (1.0 / CTX))
            pbuf[0, :] = _tree_sum(
                [tgt_rows[b, c, pl.ds(q * 16, 16)] * avg[q]
                 for q in range(D // 16)])
            rn = c * K
            for k in range(K):
                pbuf[k + 1, :] = _tree_sum(
                    [neg_rows[b, rn + k, pl.ds(q * 16, 16)] * avg[q]
                     for q in range(D // 16)])
            s = _tree_sum(
                [plsc.load_gather(pbuf, [lane_sel, col_idx[j]])
                 for j in range(16)])
            x = sgn * s
            y = jnp.exp(-jnp.abs(x))
            t = y / (y + 2.0)
            u = t * t
            poly = 1.0 + u * (1.0 / 3.0 + u * (1.0 / 5.0 + u * (1.0 / 7.0)))
            sp = jnp.maximum(-x, 0.0) + 2.0 * t * poly
            return acc + jnp.where(valid, sp, 0.0)

        return lax.fori_loop(0, C, row_body, acc0)

    fire(0, 0)
    acc = jnp.zeros((16,), jnp.float32)

    def body(i, acc):
        fire(2 * i + 1, 1)
        drain(0)
        acc = compute(0, acc)

        @pl.when(i < N_CHUNKS // 2 - 1)
        def _():
            fire(2 * i + 2, 0)

        drain(1)
        return compute(1, acc)

    acc = lax.fori_loop(0, N_CHUNKS // 2, body, acc)
    acc_buf[...] = acc
    pltpu.sync_copy(acc_buf, out_hbm.at[wid, :])


_sc_loss = functools.partial(
    pl.kernel,
    mesh=plsc.VectorSubcoreMesh(core_axis_name="c", subcore_axis_name="s"),
    compiler_params=pltpu.CompilerParams(
        needs_layout_passes=False, use_tc_tiling_on_sc=False
    ),
    out_type=jax.ShapeDtypeStruct((NW, 16), jnp.float32),
    scratch_types=[
        pltpu.VMEM((2, C), jnp.int32),
        pltpu.VMEM((2, C, LP), jnp.int32),
        pltpu.VMEM((2, C, LP), jnp.int32),
        pltpu.VMEM((2, C * CTX), jnp.int32),
        pltpu.VMEM((2, C * K), jnp.int32),
        pltpu.VMEM((2, C, D), jnp.float32),
        pltpu.VMEM((2, C * CTX, D), jnp.float32),
        pltpu.VMEM((2, C * K, D), jnp.float32),
        pltpu.VMEM((16, 16), jnp.float32),
        pltpu.VMEM((16,), jnp.float32),
        pltpu.SemaphoreType.DMA((2,)),
        pltpu.SemaphoreType.DMA((2,)),
        pltpu.SemaphoreType.DMA((2,)),
        pltpu.SemaphoreType.DMA((2,)),
    ],
)(_sc_loss_kernel)


def _tc_loss_kernel(x_ref, o_ref):
    o_ref[0, 0] = jnp.sum(x_ref[...])


_tc_loss = pl.pallas_call(
    _tc_loss_kernel,
    out_shape=jax.ShapeDtypeStruct((1, 1), jnp.float32),
    out_specs=pl.BlockSpec(memory_space=pltpu.SMEM),
)


def kernel(i_emb, o_emb, target_wids, context_wids, neg_wids):
    itab = jnp.pad(i_emb, ((0, 0), (0, LP - D))).reshape(2 * (V + 1), D)
    otab = jnp.pad(o_emb, ((0, 0), (0, LP - D))).reshape(2 * (V + 1), D)
    tgt = target_wids.astype(jnp.int32) * 2
    ctxp = jnp.pad(context_wids.astype(jnp.int32) * 2, ((0, 0), (0, LP - CTX)))
    negp = jnp.pad(neg_wids.astype(jnp.int32) * 2, ((0, 0), (0, LP - K)))
    partials = _sc_loss(itab, otab, tgt, ctxp, negp)
    loss = _tc_loss(partials)
    return loss[0, 0]


# speedup vs baseline: 1.5141x; 1.1067x over previous
import functools

import jax
import jax.numpy as jnp
from jax import lax
from jax.experimental import pallas as pl
from jax.experimental.pallas import tpu as pltpu
from jax.experimental.pallas import tpu_sc as plsc

V = 100000
D = 64
K = 10
CTX = 10
B = 16384
LP = 128
NC = 2
NS = 16
NW = NC * NS
BPW = B // NW
C = 32
N_CHUNKS = BPW // C
S = K + 1


def _tree_sum(vals):
    vals = list(vals)
    while len(vals) > 1:
        nxt = [a + b for a, b in zip(vals[0::2], vals[1::2])]
        if len(vals) % 2:
            nxt.append(vals[-1])
        vals = nxt
    return vals[0]


def _sc_loss_kernel(i_emb, o_emb, tgt_hbm, ctx_hbm, neg_hbm, out_hbm,
                    tgt_idx, ctx_idx, neg_idx,
                    tgt_rows, ctx_rows, neg_rows, pbuf, acc_buf,
                    sem_i, sem_t, sem_c, sem_n):
    wid = lax.axis_index("s") * NC + lax.axis_index("c")
    base = wid * BPW

    def fire(t, b):
        row0 = base + t * C
        sl = pl.ds(row0, C)
        pltpu.async_copy(tgt_hbm.at[sl], tgt_idx.at[b], sem_i.at[b])
        pltpu.async_copy(ctx_hbm.at[:, sl], ctx_idx.at[b], sem_i.at[b])
        pltpu.async_copy(neg_hbm.at[:, sl], neg_idx.at[b], sem_i.at[b])
        pltpu.make_async_copy(tgt_hbm.at[sl], tgt_idx.at[b], sem_i.at[b]).wait()
        pltpu.make_async_copy(
            ctx_hbm.at[:, sl], ctx_idx.at[b], sem_i.at[b]).wait()
        pltpu.make_async_copy(
            neg_hbm.at[:, sl], neg_idx.at[b], sem_i.at[b]).wait()
        for g in range(C // 16):
            gsl = pl.ds(g * 16, 16)
            tgt_idx[b, gsl] = tgt_idx[b, gsl] * 2
        for j in range(CTX):
            for g in range(C // 16):
                gsl = pl.ds(g * 16, 16)
                ctx_idx[b, j, gsl] = ctx_idx[b, j, gsl] * 2
                neg_idx[b, j, gsl] = neg_idx[b, j, gsl] * 2
        pltpu.async_copy(o_emb.at[tgt_idx.at[b]], tgt_rows.at[b], sem_t.at[b])
        for j in range(CTX):
            pltpu.async_copy(
                i_emb.at[ctx_idx.at[b, j]], ctx_rows.at[b, j], sem_c.at[b])
            pltpu.async_copy(
                o_emb.at[neg_idx.at[b, j]], neg_rows.at[b, j], sem_n.at[b])

    def drain(b):
        pltpu.make_async_copy(
            o_emb.at[tgt_idx.at[b]], tgt_rows.at[b], sem_t.at[b]).wait()
        for j in range(CTX):
            pltpu.make_async_copy(
                i_emb.at[ctx_idx.at[b, j]], ctx_rows.at[b, j],
                sem_c.at[b]).wait()
            pltpu.make_async_copy(
                o_emb.at[neg_idx.at[b, j]], neg_rows.at[b, j],
                sem_n.at[b]).wait()

    lane = lax.iota(jnp.int32, 16)
    lane_sel = jnp.where(lane < S, lane, 0)
    sgn = jnp.where(lane == 0, 1.0, -1.0)
    valid = lane < S
    col_idx = [jnp.full((16,), j, jnp.int32) for j in range(16)]

    def compute(b, acc0):
        def row_body(c, acc):
            avg = []
            for q in range(D // 16):
                a = _tree_sum(
                    [ctx_rows[b, j, c, pl.ds(q * 16, 16)]
                     for j in range(CTX)])
                avg.append(a * (1.0 / CTX))
            pbuf[0, :] = _tree_sum(
                [tgt_rows[b, c, pl.ds(q * 16, 16)] * avg[q]
                 for q in range(D // 16)])
            for k in range(K):
                pbuf[k + 1, :] = _tree_sum(
                    [neg_rows[b, k, c, pl.ds(q * 16, 16)] * avg[q]
                     for q in range(D // 16)])
            s = _tree_sum(
                [plsc.load_gather(pbuf, [lane_sel, col_idx[j]])
                 for j in range(16)])
            x = sgn * s
            y = jnp.exp(-jnp.abs(x))
            t = y / (y + 2.0)
            u = t * t
            poly = 1.0 + u * (1.0 / 3.0 + u * (1.0 / 5.0 + u * (1.0 / 7.0)))
            sp = jnp.maximum(-x, 0.0) + 2.0 * t * poly
            return acc + jnp.where(valid, sp, 0.0)

        return lax.fori_loop(0, C, row_body, acc0)

    fire(0, 0)
    acc = jnp.zeros((16,), jnp.float32)

    def body(i, acc):
        fire(2 * i + 1, 1)
        drain(0)
        acc = compute(0, acc)

        @pl.when(i < N_CHUNKS // 2 - 1)
        def _():
            fire(2 * i + 2, 0)

        drain(1)
        return compute(1, acc)

    acc = lax.fori_loop(0, N_CHUNKS // 2, body, acc)
    acc_buf[...] = acc
    pltpu.sync_copy(acc_buf, out_hbm.at[wid, :])


_sc_loss = functools.partial(
    pl.kernel,
    mesh=plsc.VectorSubcoreMesh(core_axis_name="c", subcore_axis_name="s"),
    compiler_params=pltpu.CompilerParams(
        needs_layout_passes=False, use_tc_tiling_on_sc=False
    ),
    out_type=jax.ShapeDtypeStruct((NW, 16), jnp.float32),
    scratch_types=[
        pltpu.VMEM((2, C), jnp.int32),
        pltpu.VMEM((2, CTX, C), jnp.int32),
        pltpu.VMEM((2, K, C), jnp.int32),
        pltpu.VMEM((2, C, D), jnp.float32),
        pltpu.VMEM((2, CTX, C, D), jnp.float32),
        pltpu.VMEM((2, K, C, D), jnp.float32),
        pltpu.VMEM((16, 16), jnp.float32),
        pltpu.VMEM((16,), jnp.float32),
        pltpu.SemaphoreType.DMA((2,)),
        pltpu.SemaphoreType.DMA((2,)),
        pltpu.SemaphoreType.DMA((2,)),
        pltpu.SemaphoreType.DMA((2,)),
    ],
)(_sc_loss_kernel)


def _tc_loss_kernel(x_ref, o_ref):
    o_ref[0, 0] = jnp.sum(x_ref[...])


_tc_loss = pl.pallas_call(
    _tc_loss_kernel,
    out_shape=jax.ShapeDtypeStruct((1, 1), jnp.float32),
    out_specs=pl.BlockSpec(memory_space=pltpu.SMEM),
)


def kernel(i_emb, o_emb, target_wids, context_wids, neg_wids):
    itab = jnp.pad(i_emb, ((0, 0), (0, LP - D))).reshape(2 * (V + 1), D)
    otab = jnp.pad(o_emb, ((0, 0), (0, LP - D))).reshape(2 * (V + 1), D)
    tgt = target_wids.astype(jnp.int32)
    ctxT = context_wids.astype(jnp.int32).T
    negT = neg_wids.astype(jnp.int32).T
    partials = _sc_loss(itab, otab, tgt, ctxT, negT)
    loss = _tc_loss(partials)
    return loss[0, 0]
